# Initial kernel scaffold; baseline (speedup 1.0000x reference)
import jax
import jax.numpy as jnp
from jax.experimental import pallas as pl


def kernel(fitnesses, genes):
    raise NotImplementedError("write your pallas kernel here")


# speedup vs baseline: 3.0263x; 3.0263x over previous
import functools

import jax
import jax.numpy as jnp
from jax import lax
from jax.experimental import pallas as pl
from jax.experimental.pallas import tpu as pltpu
from jax.experimental.pallas import tpu_sc as plsc

_NUM_ISLANDS = 4
_GPI = 16384
_NSEL = 4096
_DIM = 128
_R = 128
_C = 128
_SR = _NSEL // _C

_ROWS = _NUM_ISLANDS * _NSEL
_NC = 2
_NS = 16
_NW = _NC * _NS
_BPW = _ROWS // _NW
_CH = 128
_NCH = _BPW // _CH

_SIGN = -(2 ** 31)


def _topk_body(f_ref, ids_ref, w_ref):
    f = f_ref[0]
    f = jnp.where(f == 0.0, jnp.float32(0.0), f)
    bits = lax.bitcast_convert_type(f, jnp.int32)
    key = jnp.where(bits >= 0, bits, jnp.bitwise_xor(jnp.bitwise_not(bits), _SIGN))

    row = lax.broadcasted_iota(jnp.int32, (_R, _C), 0)
    col = lax.broadcasted_iota(jnp.int32, (_R, _C), 1)
    pos = row * _C + col
    idx = pos

    size = 2
    while size <= _GPI:
        d = size // 2
        while d >= 1:
            if d < _C:
                axis, sh = 1, d
            else:
                axis, sh = 0, d // _C
            n_ax = _C if axis == 1 else _R
            k_lo = pltpu.roll(key, n_ax - sh, axis)
            k_hi = pltpu.roll(key, sh, axis)
            i_lo = pltpu.roll(idx, n_ax - sh, axis)
            i_hi = pltpu.roll(idx, sh, axis)
            upper = (pos & d) != 0
            pk = jnp.where(upper, k_hi, k_lo)
            pi = jnp.where(upper, i_hi, i_lo)
            a_first = (key > pk) | ((key == pk) & (idx < pi))
            take_earlier = (~upper) == ((pos & size) == 0)
            keep_a = a_first == take_earlier
            key = jnp.where(keep_a, key, pk)
            idx = jnp.where(keep_a, idx, pi)
            d //= 2
        size *= 2

    sel_k = key[:_SR]
    sel_i = idx[:_SR]
    fb = jnp.where(sel_k >= 0, sel_k, jnp.bitwise_not(jnp.bitwise_xor(sel_k, _SIGN)))
    fsel = lax.bitcast_convert_type(fb, jnp.float32)
    m = jnp.max(fsel)
    e = jnp.exp(fsel - m)
    w = e / jnp.sum(e)
    ids_ref[0] = sel_i + pl.program_id(0) * _GPI
    w_ref[0] = w


_topk_call = pl.pallas_call(
    _topk_body,
    grid=(_NUM_ISLANDS,),
    in_specs=[pl.BlockSpec((1, _R, _C), lambda i: (i, 0, 0))],
    out_specs=[
        pl.BlockSpec((1, _SR, _C), lambda i: (i, 0, 0)),
        pl.BlockSpec((1, _SR, _C), lambda i: (i, 0, 0)),
    ],
    out_shape=[
        jax.ShapeDtypeStruct((_NUM_ISLANDS, _SR, _C), jnp.int32),
        jax.ShapeDtypeStruct((_NUM_ISLANDS, _SR, _C), jnp.float32),
    ],
)


def _gather_body(genes_hbm, ids_hbm, out_hbm, idx_v, rows_v, sem):
    wid = lax.axis_index("s") * _NC + lax.axis_index("c")
    pltpu.sync_copy(ids_hbm.at[pl.ds(wid * _NCH, _NCH)], idx_v)
    descs = []
    for j in range(_NCH):
        descs.append(
            pltpu.async_copy(
                genes_hbm.at[idx_v.at[j]],
                rows_v.at[pl.ds(j * _CH, _CH)],
                sem,
            )
        )
    for dsc in descs:
        dsc.wait()
    pltpu.sync_copy(rows_v, out_hbm.at[pl.ds(wid * _BPW, _BPW)])


@functools.cache
def _gather_call():
    return pl.kernel(
        _gather_body,
        out_type=jax.ShapeDtypeStruct((_ROWS, _DIM), jnp.float32),
        mesh=plsc.VectorSubcoreMesh(
            core_axis_name="c", subcore_axis_name="s",
            num_cores=_NC, num_subcores=_NS,
        ),
        scratch_types=[
            pltpu.VMEM((_NCH, _CH), jnp.int32),
            pltpu.VMEM((_BPW, _DIM), jnp.float32),
            pltpu.SemaphoreType.DMA,
        ],
    )

_CBLK = 512


def _norm_body(x_ref, w_ref, o_ref):
    x = x_ref[...]
    n2 = jnp.sum(x * x, axis=1, keepdims=True)
    n = jnp.maximum(jnp.sqrt(n2), 1e-12)
    o_ref[...] = x * (w_ref[...] / n)


_norm_call = pl.pallas_call(
    _norm_body,
    grid=(_ROWS // _CBLK,),
    in_specs=[
        pl.BlockSpec((_CBLK, _DIM), lambda i: (i, 0)),
        pl.BlockSpec((_CBLK, 1), lambda i: (i, 0)),
    ],
    out_specs=pl.BlockSpec((_CBLK, _DIM), lambda i: (i, 0)),
    out_shape=jax.ShapeDtypeStruct((_ROWS, _DIM), jnp.float32),
)


@jax.jit
def kernel(fitnesses, genes):
    f4 = fitnesses.reshape(_NUM_ISLANDS, _R, _C)
    ids, w = _topk_call(f4)
    ids2d = ids.reshape(_ROWS // _C, _C)
    rows = _gather_call()(genes, ids2d)
    out = _norm_call(rows, w.reshape(_ROWS, 1))
    return out.reshape(_NUM_ISLANDS, _NSEL, _DIM)


# speedup vs baseline: 3.3417x; 1.1042x over previous
import functools

import jax
import jax.numpy as jnp
from jax import lax
from jax.experimental import pallas as pl
from jax.experimental.pallas import tpu as pltpu
from jax.experimental.pallas import tpu_sc as plsc

_NUM_ISLANDS = 4
_GPI = 16384
_NSEL = 4096
_DIM = 128
_R = 128
_C = 128
_SR = _NSEL // _C

_ROWS = _NUM_ISLANDS * _NSEL
_NC = 2
_NS = 16
_NW = _NC * _NS
_BPW = _ROWS // _NW
_CH = 128
_NCH = _BPW // _CH

_SIGN = -(2 ** 31)


_AR = _NUM_ISLANDS * _R


def _topk_body(f_ref, ids_ref, w_ref):
    f = f_ref[...]
    f = jnp.where(f == 0.0, jnp.float32(0.0), f)
    bits = lax.bitcast_convert_type(f, jnp.int32)
    key = jnp.where(bits >= 0, bits, jnp.bitwise_xor(jnp.bitwise_not(bits), _SIGN))

    row = lax.broadcasted_iota(jnp.int32, (_AR, _C), 0)
    col = lax.broadcasted_iota(jnp.int32, (_AR, _C), 1)
    idx = row * _C + col
    pos = ((row & (_R - 1)) * _C) | col

    size = 2
    while size <= _GPI:
        d = size // 2
        while d >= 1:
            if d < _C:
                axis, sh, n_ax = 1, d, _C
            else:
                axis, sh, n_ax = 0, d // _C, _AR
            k_lo = pltpu.roll(key, n_ax - sh, axis)
            k_hi = pltpu.roll(key, sh, axis)
            i_lo = pltpu.roll(idx, n_ax - sh, axis)
            i_hi = pltpu.roll(idx, sh, axis)
            upper = (pos & d) != 0
            pk = jnp.where(upper, k_hi, k_lo)
            pi = jnp.where(upper, i_hi, i_lo)
            a_first = (key > pk) | ((key == pk) & (idx < pi))
            take_earlier = (~upper) == ((pos & size) == 0)
            keep_a = a_first == take_earlier
            key = jnp.where(keep_a, key, pk)
            idx = jnp.where(keep_a, idx, pi)
            d //= 2
        size *= 2

    for i in range(_NUM_ISLANDS):
        sel_k = key[i * _R:i * _R + _SR]
        fb = jnp.where(sel_k >= 0, sel_k,
                       jnp.bitwise_not(jnp.bitwise_xor(sel_k, _SIGN)))
        fsel = lax.bitcast_convert_type(fb, jnp.float32)
        e = jnp.exp(fsel - jnp.max(fsel))
        ids_ref[i] = idx[i * _R:i * _R + _SR]
        w_ref[i] = e / jnp.sum(e)


_topk_call = pl.pallas_call(
    _topk_body,
    out_shape=[
        jax.ShapeDtypeStruct((_NUM_ISLANDS, _SR, _C), jnp.int32),
        jax.ShapeDtypeStruct((_NUM_ISLANDS, _SR, _C), jnp.float32),
    ],
)


def _gather_body(genes_hbm, ids_hbm, out_hbm, idx_v, rows_v, sem):
    wid = lax.axis_index("s") * _NC + lax.axis_index("c")
    pltpu.sync_copy(ids_hbm.at[pl.ds(wid * _NCH, _NCH)], idx_v)
    descs = []
    for j in range(_NCH):
        descs.append(
            pltpu.async_copy(
                genes_hbm.at[idx_v.at[j]],
                rows_v.at[pl.ds(j * _CH, _CH)],
                sem,
            )
        )
    for dsc in descs:
        dsc.wait()
    pltpu.sync_copy(rows_v, out_hbm.at[pl.ds(wid * _BPW, _BPW)])


@functools.cache
def _gather_call():
    return pl.kernel(
        _gather_body,
        out_type=jax.ShapeDtypeStruct((_ROWS, _DIM), jnp.float32),
        mesh=plsc.VectorSubcoreMesh(
            core_axis_name="c", subcore_axis_name="s",
            num_cores=_NC, num_subcores=_NS,
        ),
        scratch_types=[
            pltpu.VMEM((_NCH, _CH), jnp.int32),
            pltpu.VMEM((_BPW, _DIM), jnp.float32),
            pltpu.SemaphoreType.DMA,
        ],
    )

_CBLK = 512


def _norm_body(x_ref, w_ref, o_ref):
    x = x_ref[...]
    n2 = jnp.sum(x * x, axis=1, keepdims=True)
    n = jnp.maximum(jnp.sqrt(n2), 1e-12)
    o_ref[...] = x * (w_ref[...] / n)


_norm_call = pl.pallas_call(
    _norm_body,
    grid=(_ROWS // _CBLK,),
    in_specs=[
        pl.BlockSpec((_CBLK, _DIM), lambda i: (i, 0)),
        pl.BlockSpec((_CBLK, 1), lambda i: (i, 0)),
    ],
    out_specs=pl.BlockSpec((_CBLK, _DIM), lambda i: (i, 0)),
    out_shape=jax.ShapeDtypeStruct((_ROWS, _DIM), jnp.float32),
)


@jax.jit
def kernel(fitnesses, genes):
    f2 = fitnesses.reshape(_AR, _C)
    ids, w = _topk_call(f2)
    ids2d = ids.reshape(_ROWS // _C, _C)
    rows = _gather_call()(genes, ids2d)
    out = _norm_call(rows, w.reshape(_ROWS, 1))
    return out.reshape(_NUM_ISLANDS, _NSEL, _DIM)


# speedup vs baseline: 3.5287x; 1.0560x over previous
import functools

import jax
import jax.numpy as jnp
from jax import lax
from jax.experimental import pallas as pl
from jax.experimental.pallas import tpu as pltpu
from jax.experimental.pallas import tpu_sc as plsc

_NUM_ISLANDS = 4
_GPI = 16384
_NSEL = 4096
_DIM = 128
_R = 128
_C = 128
_SR = _NSEL // _C

_ROWS = _NUM_ISLANDS * _NSEL
_NC = 2
_NS = 16
_NW = _NC * _NS
_BPW = _ROWS // _NW
_CH = 128
_NCH = _BPW // _CH

_SIGN = -(2 ** 31)


_AR = _NUM_ISLANDS * _R


def _topk_body(f_ref, ids_ref, w_ref):
    f = f_ref[...]
    f = jnp.where(f == 0.0, jnp.float32(0.0), f)
    bits = lax.bitcast_convert_type(f, jnp.int32)
    key = jnp.where(bits >= 0, bits, jnp.bitwise_xor(jnp.bitwise_not(bits), _SIGN))

    row = lax.broadcasted_iota(jnp.int32, (_AR, _C), 0)
    col = lax.broadcasted_iota(jnp.int32, (_AR, _C), 1)
    pos = col * _C + (row & (_R - 1))
    idx = (row >> 7) * _GPI + pos

    size = 2
    while size <= _GPI:
        d = size // 2
        while d >= 1:
            if d < _R:
                axis, sh, n_ax = 0, d, _AR
            else:
                axis, sh, n_ax = 1, d // _R, _C
            k_lo = pltpu.roll(key, n_ax - sh, axis)
            k_hi = pltpu.roll(key, sh, axis)
            i_lo = pltpu.roll(idx, n_ax - sh, axis)
            i_hi = pltpu.roll(idx, sh, axis)
            upper = (pos & d) != 0
            pk = jnp.where(upper, k_hi, k_lo)
            pi = jnp.where(upper, i_hi, i_lo)
            a_first = (key > pk) | ((key == pk) & (idx < pi))
            take_earlier = (~upper) == ((pos & size) == 0)
            keep_a = a_first == take_earlier
            key = jnp.where(keep_a, key, pk)
            idx = jnp.where(keep_a, idx, pi)
            d //= 2
        size *= 2

    for i in range(_NUM_ISLANDS):
        sel_k = key[i * _R:(i + 1) * _R, :_SR].T
        fb = jnp.where(sel_k >= 0, sel_k,
                       jnp.bitwise_not(jnp.bitwise_xor(sel_k, _SIGN)))
        fsel = lax.bitcast_convert_type(fb, jnp.float32)
        e = jnp.exp(fsel - jnp.max(fsel))
        ids_ref[i] = idx[i * _R:(i + 1) * _R, :_SR].T
        w_ref[i] = e / jnp.sum(e)


_topk_call = pl.pallas_call(
    _topk_body,
    out_shape=[
        jax.ShapeDtypeStruct((_NUM_ISLANDS, _SR, _C), jnp.int32),
        jax.ShapeDtypeStruct((_NUM_ISLANDS, _SR, _C), jnp.float32),
    ],
)


def _gather_body(genes_hbm, ids_hbm, out_hbm, idx_v, rows_v, sem):
    wid = lax.axis_index("s") * _NC + lax.axis_index("c")
    pltpu.sync_copy(ids_hbm.at[pl.ds(wid * _NCH, _NCH)], idx_v)
    descs = []
    for j in range(_NCH):
        descs.append(
            pltpu.async_copy(
                genes_hbm.at[idx_v.at[j]],
                rows_v.at[pl.ds(j * _CH, _CH)],
                sem,
            )
        )
    for dsc in descs:
        dsc.wait()
    pltpu.sync_copy(rows_v, out_hbm.at[pl.ds(wid * _BPW, _BPW)])


@functools.cache
def _gather_call():
    return pl.kernel(
        _gather_body,
        out_type=jax.ShapeDtypeStruct((_ROWS, _DIM), jnp.float32),
        mesh=plsc.VectorSubcoreMesh(
            core_axis_name="c", subcore_axis_name="s",
            num_cores=_NC, num_subcores=_NS,
        ),
        scratch_types=[
            pltpu.VMEM((_NCH, _CH), jnp.int32),
            pltpu.VMEM((_BPW, _DIM), jnp.float32),
            pltpu.SemaphoreType.DMA,
        ],
    )

_CBLK = 512


def _norm_body(x_ref, w_ref, o_ref):
    x = x_ref[...]
    n2 = jnp.sum(x * x, axis=1, keepdims=True)
    n = jnp.maximum(jnp.sqrt(n2), 1e-12)
    o_ref[...] = x * (w_ref[...] / n)


_norm_call = pl.pallas_call(
    _norm_body,
    grid=(_ROWS // _CBLK,),
    in_specs=[
        pl.BlockSpec((_CBLK, _DIM), lambda i: (i, 0)),
        pl.BlockSpec((_CBLK, 1), lambda i: (i, 0)),
    ],
    out_specs=pl.BlockSpec((_CBLK, _DIM), lambda i: (i, 0)),
    out_shape=jax.ShapeDtypeStruct((_ROWS, _DIM), jnp.float32),
)


@jax.jit
def kernel(fitnesses, genes):
    f2 = (
        fitnesses.reshape(_NUM_ISLANDS, _R, _C)
        .transpose(0, 2, 1)
        .reshape(_AR, _C)
    )
    ids, w = _topk_call(f2)
    ids2d = ids.reshape(_ROWS // _C, _C)
    rows = _gather_call()(genes, ids2d)
    out = _norm_call(rows, w.reshape(_ROWS, 1))
    return out.reshape(_NUM_ISLANDS, _NSEL, _DIM)


# speedup vs baseline: 3.5637x; 1.0099x over previous
import functools

import jax
import jax.numpy as jnp
from jax import lax
from jax.experimental import pallas as pl
from jax.experimental.pallas import tpu as pltpu
from jax.experimental.pallas import tpu_sc as plsc

_NUM_ISLANDS = 4
_GPI = 16384
_NSEL = 4096
_DIM = 128
_R = 128
_C = 128
_SR = _NSEL // _C

_ROWS = _NUM_ISLANDS * _NSEL
_NC = 2
_NS = 16
_NW = _NC * _NS
_BPW = _ROWS // _NW
_CH = 128
_NCH = _BPW // _CH

_SIGN = -(2 ** 31)


_AR = _NUM_ISLANDS * _R


def _topk_body(f_ref, ids_ref, w_ref):
    f = f_ref[...]
    f = jnp.where(f == 0.0, jnp.float32(0.0), f)
    bits = lax.bitcast_convert_type(f, jnp.int32)
    key = jnp.where(bits >= 0, bits, jnp.bitwise_xor(jnp.bitwise_not(bits), _SIGN))

    row = lax.broadcasted_iota(jnp.int32, (_AR, _C), 0)
    col = lax.broadcasted_iota(jnp.int32, (_AR, _C), 1)
    pos = col * _C + (row & (_R - 1))
    idx = (row >> 7) * _GPI + pos

    size = 2
    while size <= _GPI:
        d = size // 2
        while d >= 1:
            upper = (pos & d) != 0
            if d < _R:
                g = _AR // (2 * d)

                def _xchg(x, g=g, d=d):
                    xr = x.reshape(g, 2, d, _C)
                    return jnp.concatenate(
                        [xr[:, 1:2], xr[:, 0:1]], axis=1
                    ).reshape(_AR, _C)

                pk = _xchg(key)
                pi = _xchg(idx)
            else:
                sh = d // _R
                k_lo = pltpu.roll(key, _C - sh, 1)
                k_hi = pltpu.roll(key, sh, 1)
                i_lo = pltpu.roll(idx, _C - sh, 1)
                i_hi = pltpu.roll(idx, sh, 1)
                pk = jnp.where(upper, k_hi, k_lo)
                pi = jnp.where(upper, i_hi, i_lo)
            a_first = (key > pk) | ((key == pk) & (idx < pi))
            take_earlier = (~upper) == ((pos & size) == 0)
            keep_a = a_first == take_earlier
            key = jnp.where(keep_a, key, pk)
            idx = jnp.where(keep_a, idx, pi)
            d //= 2
        size *= 2

    for i in range(_NUM_ISLANDS):
        sel_k = key[i * _R:(i + 1) * _R, :_SR].T
        fb = jnp.where(sel_k >= 0, sel_k,
                       jnp.bitwise_not(jnp.bitwise_xor(sel_k, _SIGN)))
        fsel = lax.bitcast_convert_type(fb, jnp.float32)
        e = jnp.exp(fsel - jnp.max(fsel))
        ids_ref[i] = idx[i * _R:(i + 1) * _R, :_SR].T
        w_ref[i] = e / jnp.sum(e)


_topk_call = pl.pallas_call(
    _topk_body,
    out_shape=[
        jax.ShapeDtypeStruct((_NUM_ISLANDS, _SR, _C), jnp.int32),
        jax.ShapeDtypeStruct((_NUM_ISLANDS, _SR, _C), jnp.float32),
    ],
)


def _gather_body(genes_hbm, ids_hbm, out_hbm, idx_v, rows_v, sem):
    wid = lax.axis_index("s") * _NC + lax.axis_index("c")
    pltpu.sync_copy(ids_hbm.at[pl.ds(wid * _NCH, _NCH)], idx_v)
    descs = []
    for j in range(_NCH):
        descs.append(
            pltpu.async_copy(
                genes_hbm.at[idx_v.at[j]],
                rows_v.at[pl.ds(j * _CH, _CH)],
                sem,
            )
        )
    for dsc in descs:
        dsc.wait()
    pltpu.sync_copy(rows_v, out_hbm.at[pl.ds(wid * _BPW, _BPW)])


@functools.cache
def _gather_call():
    return pl.kernel(
        _gather_body,
        out_type=jax.ShapeDtypeStruct((_ROWS, _DIM), jnp.float32),
        mesh=plsc.VectorSubcoreMesh(
            core_axis_name="c", subcore_axis_name="s",
            num_cores=_NC, num_subcores=_NS,
        ),
        scratch_types=[
            pltpu.VMEM((_NCH, _CH), jnp.int32),
            pltpu.VMEM((_BPW, _DIM), jnp.float32),
            pltpu.SemaphoreType.DMA,
        ],
    )

_CBLK = 512


def _norm_body(x_ref, w_ref, o_ref):
    x = x_ref[...]
    n2 = jnp.sum(x * x, axis=1, keepdims=True)
    n = jnp.maximum(jnp.sqrt(n2), 1e-12)
    o_ref[...] = x * (w_ref[...] / n)


_norm_call = pl.pallas_call(
    _norm_body,
    grid=(_ROWS // _CBLK,),
    in_specs=[
        pl.BlockSpec((_CBLK, _DIM), lambda i: (i, 0)),
        pl.BlockSpec((_CBLK, 1), lambda i: (i, 0)),
    ],
    out_specs=pl.BlockSpec((_CBLK, _DIM), lambda i: (i, 0)),
    out_shape=jax.ShapeDtypeStruct((_ROWS, _DIM), jnp.float32),
)


@jax.jit
def kernel(fitnesses, genes):
    f2 = (
        fitnesses.reshape(_NUM_ISLANDS, _R, _C)
        .transpose(0, 2, 1)
        .reshape(_AR, _C)
    )
    ids, w = _topk_call(f2)
    ids2d = ids.reshape(_ROWS // _C, _C)
    rows = _gather_call()(genes, ids2d)
    out = _norm_call(rows, w.reshape(_ROWS, 1))
    return out.reshape(_NUM_ISLANDS, _NSEL, _DIM)


# speedup vs baseline: 3.7064x; 1.0401x over previous
import functools

import jax
import jax.numpy as jnp
from jax import lax
from jax.experimental import pallas as pl
from jax.experimental.pallas import tpu as pltpu
from jax.experimental.pallas import tpu_sc as plsc

_NUM_ISLANDS = 4
_GPI = 16384
_NSEL = 4096
_DIM = 128
_R = 128
_C = 128
_SR = _NSEL // _C

_ROWS = _NUM_ISLANDS * _NSEL
_NC = 2
_NS = 16
_NW = _NC * _NS
_BPW = _ROWS // _NW
_CH = 128
_NCH = _BPW // _CH

_SIGN = -(2 ** 31)


_AR = _NUM_ISLANDS * _R


def _topk_body(f_ref, ids_ref, w_ref):
    f = jnp.concatenate(
        [f_ref[i * _R:(i + 1) * _R, :].T for i in range(_NUM_ISLANDS)], axis=0
    )
    f = jnp.where(f == 0.0, jnp.float32(0.0), f)
    bits = lax.bitcast_convert_type(f, jnp.int32)
    key = jnp.where(bits >= 0, bits, jnp.bitwise_xor(jnp.bitwise_not(bits), _SIGN))

    row = lax.broadcasted_iota(jnp.int32, (_AR, _C), 0)
    col = lax.broadcasted_iota(jnp.int32, (_AR, _C), 1)
    pos = col * _C + (row & (_R - 1))
    idx = (row >> 7) * _GPI + pos

    size = 2
    while size <= _GPI:
        d = size // 2
        while d >= 1:
            upper = (pos & d) != 0
            if d < _R:
                g = _AR // (2 * d)

                def _xchg(x, g=g, d=d):
                    xr = x.reshape(g, 2, d, _C)
                    return jnp.concatenate(
                        [xr[:, 1:2], xr[:, 0:1]], axis=1
                    ).reshape(_AR, _C)

                pk = _xchg(key)
                pi = _xchg(idx)
            else:
                sh = d // _R
                k_lo = pltpu.roll(key, _C - sh, 1)
                k_hi = pltpu.roll(key, sh, 1)
                i_lo = pltpu.roll(idx, _C - sh, 1)
                i_hi = pltpu.roll(idx, sh, 1)
                pk = jnp.where(upper, k_hi, k_lo)
                pi = jnp.where(upper, i_hi, i_lo)
            a_first = (key > pk) | ((key == pk) & (idx < pi))
            take_earlier = (~upper) == ((pos & size) == 0)
            keep_a = a_first == take_earlier
            key = jnp.where(keep_a, key, pk)
            idx = jnp.where(keep_a, idx, pi)
            d //= 2
        size *= 2

    for i in range(_NUM_ISLANDS):
        sel_k = key[i * _R:(i + 1) * _R, :_SR].T
        fb = jnp.where(sel_k >= 0, sel_k,
                       jnp.bitwise_not(jnp.bitwise_xor(sel_k, _SIGN)))
        fsel = lax.bitcast_convert_type(fb, jnp.float32)
        e = jnp.exp(fsel - jnp.max(fsel))
        ids_ref[i] = idx[i * _R:(i + 1) * _R, :_SR].T
        w_ref[i] = e / jnp.sum(e)


_topk_call = pl.pallas_call(
    _topk_body,
    out_shape=[
        jax.ShapeDtypeStruct((_NUM_ISLANDS, _SR, _C), jnp.int32),
        jax.ShapeDtypeStruct((_NUM_ISLANDS, _SR, _C), jnp.float32),
    ],
)


def _gather_body(genes_hbm, ids_hbm, out_hbm, idx_v, rows_v, sem):
    wid = lax.axis_index("s") * _NC + lax.axis_index("c")
    pltpu.sync_copy(ids_hbm.at[pl.ds(wid * _NCH, _NCH)], idx_v)
    descs = []
    for j in range(_NCH):
        descs.append(
            pltpu.async_copy(
                genes_hbm.at[idx_v.at[j]],
                rows_v.at[pl.ds(j * _CH, _CH)],
                sem,
            )
        )
    for dsc in descs:
        dsc.wait()
    pltpu.sync_copy(rows_v, out_hbm.at[pl.ds(wid * _BPW, _BPW)])


@functools.cache
def _gather_call():
    return pl.kernel(
        _gather_body,
        out_type=jax.ShapeDtypeStruct((_ROWS, _DIM), jnp.float32),
        mesh=plsc.VectorSubcoreMesh(
            core_axis_name="c", subcore_axis_name="s",
            num_cores=_NC, num_subcores=_NS,
        ),
        scratch_types=[
            pltpu.VMEM((_NCH, _CH), jnp.int32),
            pltpu.VMEM((_BPW, _DIM), jnp.float32),
            pltpu.SemaphoreType.DMA,
        ],
    )

_CBLK = 512


def _norm_body(x_ref, w_ref, o_ref):
    x = x_ref[...]
    n2 = jnp.sum(x * x, axis=1, keepdims=True)
    n = jnp.maximum(jnp.sqrt(n2), 1e-12)
    o_ref[...] = x * (w_ref[...] / n)


_norm_call = pl.pallas_call(
    _norm_body,
    grid=(_ROWS // _CBLK,),
    in_specs=[
        pl.BlockSpec((_CBLK, _DIM), lambda i: (i, 0)),
        pl.BlockSpec((_CBLK, 1), lambda i: (i, 0)),
    ],
    out_specs=pl.BlockSpec((_CBLK, _DIM), lambda i: (i, 0)),
    out_shape=jax.ShapeDtypeStruct((_ROWS, _DIM), jnp.float32),
)


@jax.jit
def kernel(fitnesses, genes):
    ids, w = _topk_call(fitnesses.reshape(_AR, _C))
    ids2d = ids.reshape(_ROWS // _C, _C)
    rows = _gather_call()(genes, ids2d)
    out = _norm_call(rows, w.reshape(_ROWS, 1))
    return out.reshape(_NUM_ISLANDS, _NSEL, _DIM)


# speedup vs baseline: 4.3564x; 1.1753x over previous
import functools

import jax
import jax.numpy as jnp
from jax import lax
from jax.experimental import pallas as pl
from jax.experimental.pallas import tpu as pltpu
from jax.experimental.pallas import tpu_sc as plsc

_NUM_ISLANDS = 4
_GPI = 16384
_NSEL = 4096
_DIM = 128
_R = 128
_C = 128
_SR = _NSEL // _C

_ROWS = _NUM_ISLANDS * _NSEL
_NC = 2
_NS = 16
_NW = _NC * _NS
_BPW = _ROWS // _NW
_CH = 128
_NCH = _BPW // _CH

_SIGN = -(2 ** 31)


_AR = _NUM_ISLANDS * _R


def _topk_body(f_ref, ids_ref, w_ref):
    f = jnp.concatenate(
        [f_ref[i * _R:(i + 1) * _R, :].T for i in range(_NUM_ISLANDS)], axis=0
    )
    f = jnp.where(f == 0.0, jnp.float32(0.0), f)
    bits = lax.bitcast_convert_type(f, jnp.int32)
    key = jnp.where(bits >= 0, bits, jnp.bitwise_xor(jnp.bitwise_not(bits), _SIGN))

    row = lax.broadcasted_iota(jnp.int32, (_AR, _C), 0)
    col = lax.broadcasted_iota(jnp.int32, (_AR, _C), 1)
    pos = col * _C + (row & (_R - 1))
    idx = (row >> 7) * _GPI + pos

    size = 2
    while size <= _GPI:
        d = size // 2
        while d >= 1:
            upper = (pos & d) != 0
            if d < _R:
                g = _AR // (2 * d)

                def _xchg(x, g=g, d=d):
                    xr = x.reshape(g, 2, d, _C)
                    return jnp.concatenate(
                        [xr[:, 1:2], xr[:, 0:1]], axis=1
                    ).reshape(_AR, _C)

                pk = _xchg(key)
                pi = _xchg(idx)
            else:
                sh = d // _R
                k_lo = pltpu.roll(key, _C - sh, 1)
                k_hi = pltpu.roll(key, sh, 1)
                i_lo = pltpu.roll(idx, _C - sh, 1)
                i_hi = pltpu.roll(idx, sh, 1)
                pk = jnp.where(upper, k_hi, k_lo)
                pi = jnp.where(upper, i_hi, i_lo)
            a_first = (key > pk) | ((key == pk) & (idx < pi))
            take_earlier = (~upper) == ((pos & size) == 0)
            keep_a = a_first == take_earlier
            key = jnp.where(keep_a, key, pk)
            idx = jnp.where(keep_a, idx, pi)
            d //= 2
        size *= 2

    for i in range(_NUM_ISLANDS):
        sel_k = key[i * _R:(i + 1) * _R, :_SR].T
        fb = jnp.where(sel_k >= 0, sel_k,
                       jnp.bitwise_not(jnp.bitwise_xor(sel_k, _SIGN)))
        fsel = lax.bitcast_convert_type(fb, jnp.float32)
        e = jnp.exp(fsel - jnp.max(fsel))
        ids_ref[i] = idx[i * _R:(i + 1) * _R, :_SR].T
        w_ref[i] = e / jnp.sum(e)


_topk_call = pl.pallas_call(
    _topk_body,
    out_shape=[
        jax.ShapeDtypeStruct((_NUM_ISLANDS, _SR, _C), jnp.int32),
        jax.ShapeDtypeStruct((_NUM_ISLANDS, _SR, _C), jnp.float32),
    ],
)


def _gather_body(genes_hbm, ids_hbm, out_hbm, idx_v, rows_v, sem):
    wid = lax.axis_index("s") * _NC + lax.axis_index("c")
    pltpu.sync_copy(ids_hbm.at[pl.ds(wid * _NCH, _NCH)], idx_v)
    descs = []
    for j in range(_NCH):
        descs.append(
            pltpu.async_copy(
                genes_hbm.at[idx_v.at[j]],
                rows_v.at[pl.ds(j * _CH, _CH)],
                sem,
            )
        )
    for dsc in descs:
        dsc.wait()
    pltpu.sync_copy(rows_v, out_hbm.at[pl.ds(wid * _BPW, _BPW)])


@functools.cache
def _gather_call():
    return pl.kernel(
        _gather_body,
        out_type=jax.ShapeDtypeStruct((_ROWS, _DIM), jnp.float32),
        mesh=plsc.VectorSubcoreMesh(
            core_axis_name="c", subcore_axis_name="s",
            num_cores=_NC, num_subcores=_NS,
        ),
        scratch_types=[
            pltpu.VMEM((_NCH, _CH), jnp.int32),
            pltpu.VMEM((_BPW, _DIM), jnp.float32),
            pltpu.SemaphoreType.DMA,
        ],
    )

_CBLK = 2048


def _norm_body(x_ref, w_ref, o_ref):
    x = x_ref[...]
    n2 = jnp.sum(x * x, axis=1, keepdims=True)
    n = jnp.maximum(jnp.sqrt(n2), 1e-12)
    o_ref[...] = x * (w_ref[...] / n)


_norm_call = pl.pallas_call(
    _norm_body,
    grid=(_ROWS // _CBLK,),
    in_specs=[
        pl.BlockSpec((_CBLK, _DIM), lambda i: (i, 0)),
        pl.BlockSpec((_CBLK, 1), lambda i: (i, 0)),
    ],
    out_specs=pl.BlockSpec((_CBLK, _DIM), lambda i: (i, 0)),
    out_shape=jax.ShapeDtypeStruct((_ROWS, _DIM), jnp.float32),
)


@jax.jit
def kernel(fitnesses, genes):
    ids, w = _topk_call(fitnesses.reshape(_AR, _C))
    ids2d = ids.reshape(_ROWS // _C, _C)
    rows = _gather_call()(genes, ids2d)
    out = _norm_call(rows, w.reshape(_ROWS, 1))
    return out.reshape(_NUM_ISLANDS, _NSEL, _DIM)


# speedup vs baseline: 4.4502x; 1.0215x over previous
import functools

import jax
import jax.numpy as jnp
from jax import lax
from jax.experimental import pallas as pl
from jax.experimental.pallas import tpu as pltpu
from jax.experimental.pallas import tpu_sc as plsc

_NUM_ISLANDS = 4
_GPI = 16384
_NSEL = 4096
_DIM = 128
_R = 128
_C = 128
_SR = _NSEL // _C

_ROWS = _NUM_ISLANDS * _NSEL
_NC = 2
_NS = 16
_NW = _NC * _NS
_BPW = _ROWS // _NW
_CH = 128
_NCH = _BPW // _CH

_SIGN = -(2 ** 31)


_AR = _NUM_ISLANDS * _R


def _topk_body(f_ref, ids_ref, w_ref):
    f = jnp.concatenate(
        [f_ref[i * _R:(i + 1) * _R, :].T for i in range(_NUM_ISLANDS)], axis=0
    )
    f = jnp.where(f == 0.0, jnp.float32(0.0), f)
    bits = lax.bitcast_convert_type(f, jnp.int32)
    key = jnp.where(bits >= 0, bits, jnp.bitwise_xor(jnp.bitwise_not(bits), _SIGN))

    row = lax.broadcasted_iota(jnp.int32, (_AR, _C), 0)
    col = lax.broadcasted_iota(jnp.int32, (_AR, _C), 1)
    pos = col * _C + (row & (_R - 1))
    idx = (row >> 7) * _GPI + pos

    size = 2
    while size <= _GPI:
        d = size // 2
        while d >= 1:
            upper = (pos & d) != 0
            if d < _R:
                g = _AR // (2 * d)

                def _xchg(x, g=g, d=d):
                    xr = x.reshape(g, 2, d, _C)
                    return jnp.concatenate(
                        [xr[:, 1:2], xr[:, 0:1]], axis=1
                    ).reshape(_AR, _C)

                pk = _xchg(key)
                pi = _xchg(idx)
            else:
                sh = d // _R
                k_lo = pltpu.roll(key, _C - sh, 1)
                k_hi = pltpu.roll(key, sh, 1)
                i_lo = pltpu.roll(idx, _C - sh, 1)
                i_hi = pltpu.roll(idx, sh, 1)
                pk = jnp.where(upper, k_hi, k_lo)
                pi = jnp.where(upper, i_hi, i_lo)
            a_first = (key > pk) | ((key == pk) & (idx < pi))
            take_earlier = (~upper) == ((pos & size) == 0)
            keep_a = a_first == take_earlier
            key = jnp.where(keep_a, key, pk)
            idx = jnp.where(keep_a, idx, pi)
            d //= 2
        size *= 2

    for i in range(_NUM_ISLANDS):
        sel_k = key[i * _R:(i + 1) * _R, :_SR].T
        fb = jnp.where(sel_k >= 0, sel_k,
                       jnp.bitwise_not(jnp.bitwise_xor(sel_k, _SIGN)))
        fsel = lax.bitcast_convert_type(fb, jnp.float32)
        e = jnp.exp(fsel - jnp.max(fsel))
        ids_ref[i] = idx[i * _R:(i + 1) * _R, :_SR].T
        w_ref[i] = e / jnp.sum(e)


_topk_call = pl.pallas_call(
    _topk_body,
    out_shape=[
        jax.ShapeDtypeStruct((_NUM_ISLANDS, _SR, _C), jnp.int32),
        jax.ShapeDtypeStruct((_NUM_ISLANDS, _SR, _C), jnp.float32),
    ],
)


def _gather_body(genes_hbm, ids_hbm, out_hbm, idx_v, rows_v, sem):
    wid = lax.axis_index("s") * _NC + lax.axis_index("c")
    pltpu.sync_copy(ids_hbm.at[pl.ds(wid * _NCH, _NCH)], idx_v)
    descs = []
    for j in range(_NCH):
        descs.append(
            pltpu.async_copy(
                genes_hbm.at[idx_v.at[j]],
                rows_v.at[pl.ds(j * _CH, _CH)],
                sem,
            )
        )
    for dsc in descs:
        dsc.wait()
    pltpu.sync_copy(rows_v, out_hbm.at[pl.ds(wid * _BPW, _BPW)])


@functools.cache
def _gather_call():
    return pl.kernel(
        _gather_body,
        out_type=jax.ShapeDtypeStruct((_ROWS, _DIM), jnp.float32),
        mesh=plsc.VectorSubcoreMesh(
            core_axis_name="c", subcore_axis_name="s",
            num_cores=_NC, num_subcores=_NS,
        ),
        scratch_types=[
            pltpu.VMEM((_NCH, _CH), jnp.int32),
            pltpu.VMEM((_BPW, _DIM), jnp.float32),
            pltpu.SemaphoreType.DMA,
        ],
    )

_CBLK = 4096


def _norm_body(x_ref, w_ref, o_ref):
    x = x_ref[...]
    n2 = jnp.sum(x * x, axis=1, keepdims=True)
    n = jnp.maximum(jnp.sqrt(n2), 1e-12)
    o_ref[...] = x * (w_ref[...] / n)


_norm_call = pl.pallas_call(
    _norm_body,
    grid=(_ROWS // _CBLK,),
    in_specs=[
        pl.BlockSpec((_CBLK, _DIM), lambda i: (i, 0)),
        pl.BlockSpec((_CBLK, 1), lambda i: (i, 0)),
    ],
    out_specs=pl.BlockSpec((_CBLK, _DIM), lambda i: (i, 0)),
    out_shape=jax.ShapeDtypeStruct((_ROWS, _DIM), jnp.float32),
)


@jax.jit
def kernel(fitnesses, genes):
    ids, w = _topk_call(fitnesses.reshape(_AR, _C))
    ids2d = ids.reshape(_ROWS // _C, _C)
    rows = _gather_call()(genes, ids2d)
    out = _norm_call(rows, w.reshape(_ROWS, 1))
    return out.reshape(_NUM_ISLANDS, _NSEL, _DIM)
